# baseline (speedup 1.0000x reference)
import jax
import jax.numpy as jnp
from jax import lax
from jax.experimental import pallas as pl
from jax.experimental.pallas import tpu as pltpu
from jax.experimental.pallas import tpu_sc as plsc

T = 2048
D = 1024
DH = D // 2
E = 64
KTOP = 2
F = 512
FS = 1024
CAP = 256
TK = T * KTOP

NW = 32
CHUNK = TK // NW
TPW = T // NW
BM = 128
BM_LOG = 7
NB = CAP // BM
TBMAX = E + TK // BM
TRASH = TBMAX * BM
NBUF = (TBMAX + 1) * BM
BMS = 256

_HIMASK = -65536


def _pack_halves(a, b):
  lo = pltpu.bitcast(a.astype(jnp.bfloat16).astype(jnp.float32), jnp.int32)
  hi = pltpu.bitcast(b.astype(jnp.bfloat16).astype(jnp.float32), jnp.int32)
  return (hi & _HIMASK) | lax.shift_right_logical(lo, 16)


def _unpack_halves(w):
  lo = pltpu.bitcast(lax.shift_left(w, 16), jnp.float32)
  hi = pltpu.bitcast(w & _HIMASK, jnp.float32)
  return lo.astype(jnp.bfloat16), hi.astype(jnp.bfloat16)


def _router_body(x_ref, gw_ref, w_ref, i_ref, xp_ref):
  x = x_ref[...]
  gw = gw_ref[...]
  xp_ref[...] = _pack_halves(x[:, :DH], x[:, DH:])
  logits = lax.dot_general(x, gw, (((1,), (1,)), ((), ())),
                           preferred_element_type=jnp.float32)
  s = jax.nn.sigmoid(logits)
  col = lax.broadcasted_iota(jnp.int32, s.shape, 1)
  m1 = jnp.max(s, axis=1, keepdims=True)
  i1 = jnp.min(jnp.where(s == m1, col, E), axis=1, keepdims=True)
  s2 = jnp.where(col == i1, -1e30, s)
  m2 = jnp.max(s2, axis=1, keepdims=True)
  i2 = jnp.min(jnp.where(s2 == m2, col, E), axis=1, keepdims=True)
  tot = m1 + m2
  w_ref[...] = jnp.concatenate([m1 / tot, m2 / tot], axis=1)
  i_ref[...] = jnp.concatenate([i1, i2], axis=1)


def _router(x, gate_w):
  return pl.pallas_call(
      _router_body,
      out_shape=[jax.ShapeDtypeStruct((T, KTOP), jnp.float32),
                 jax.ShapeDtypeStruct((T, KTOP), jnp.int32),
                 jax.ShapeDtypeStruct((T, DH), jnp.int32)],
  )(x, gate_w)


def _scan_base():
  c0, _ = plsc.scan_count(jnp.zeros((16,), jnp.int32))
  return jnp.min(c0)


def _dispatch_body(ids_hbm, w_hbm, x_hbm,
                   buf_hbm, d0_hbm, d1_hbm, wf0_hbm, wf1_hbm, counts_hbm,
                   pf_hbm,
                   allids_v, blkb_v, pf_v, rank_v, wv_v,
                   dstg_v, dsts_v, d0_v, d1_v, wf0_v, wf1_v,
                   s0_v, s1_v, rows_v, counts_v, sem):
  wid = lax.axis_index("s") * 2 + lax.axis_index("c")
  base = wid * CHUNK
  tbase = wid * TPW
  pltpu.sync_copy(ids_hbm, allids_v)
  pltpu.sync_copy(w_hbm.at[pl.ds(base, CHUNK)], wv_v)

  for t in range(E // 16):
    counts_v[pl.ds(t * 16, 16)] = jnp.zeros((16,), jnp.int32)
  cbase = _scan_base()
  for w2 in range(NW):
    for g in range(CHUNK // 16):
      ids = allids_v[pl.ds(w2 * CHUNK + g * 16, 16)]
      c, last = plsc.scan_count(ids)
      prior = c - cbase
      cnt = plsc.load_gather(counts_v, [ids])
      rank = cnt + prior

      @pl.when(w2 == wid)
      def _(rank=rank, g=g):
        rank_v[pl.ds(g * 16, 16)] = rank

      plsc.store_scatter(counts_v, [ids], rank + 1, mask=last)

  carry = jnp.int32(0)
  for eg in range(E // 16):
    tot = counts_v[pl.ds(eg * 16, 16)]
    act = jnp.maximum(
        1, lax.shift_right_logical(jnp.minimum(tot, CAP) + (BM - 1), BM_LOG))
    inc = plsc.cumsum(act)
    excl = inc - act + carry
    blkb_v[pl.ds(eg * 16, 16)] = excl
    carry = carry + jnp.max(inc)
    epos = (lax.iota(jnp.int32, 16) + eg * 16) * NB
    plsc.store_scatter(pf_v, [epos], excl)
    plsc.store_scatter(pf_v, [epos + 1], excl + jnp.minimum(act - 1, 1))

  @pl.when(wid == NW - 1)
  def _():
    pltpu.sync_copy(counts_v, counts_hbm)
    pltpu.sync_copy(pf_v, pf_hbm)

  for g in range(CHUNK // 16):
    ids = allids_v[pl.ds(base + g * 16, 16)]
    rank = rank_v[pl.ds(g * 16, 16)]
    valid = rank < CAP
    rbase = plsc.load_gather(blkb_v, [ids]) * BM
    dstg_v[pl.ds(g * 16, 16)] = rbase + jnp.where(valid, rank, 0)
    dsts_v[pl.ds(g * 16, 16)] = jnp.where(valid, rbase + rank, TRASH)
    wf = jnp.where(valid, wv_v[pl.ds(g * 16, 16)], 0.0)
    wv_v[pl.ds(g * 16, 16)] = wf

  for g in range(TPW // 16):
    pidx = (lax.iota(jnp.int32, 16) + g * 16) * 2
    d0_v[pl.ds(g * 16, 16)] = plsc.load_gather(dstg_v, [pidx])
    d1_v[pl.ds(g * 16, 16)] = plsc.load_gather(dstg_v, [pidx + 1])
    wf0_v[pl.ds(g * 16, 16)] = plsc.load_gather(wv_v, [pidx])
    wf1_v[pl.ds(g * 16, 16)] = plsc.load_gather(wv_v, [pidx + 1])
    s0_v[pl.ds(g * 16, 16)] = plsc.load_gather(dsts_v, [pidx])
    s1_v[pl.ds(g * 16, 16)] = plsc.load_gather(dsts_v, [pidx + 1])
  pltpu.sync_copy(d0_v, d0_hbm.at[pl.ds(tbase, TPW)])
  pltpu.sync_copy(d1_v, d1_hbm.at[pl.ds(tbase, TPW)])
  pltpu.sync_copy(wf0_v, wf0_hbm.at[pl.ds(tbase, TPW)])
  pltpu.sync_copy(wf1_v, wf1_hbm.at[pl.ds(tbase, TPW)])

  pltpu.sync_copy(x_hbm.at[pl.ds(tbase, TPW)], rows_v)
  c0 = pltpu.async_copy(rows_v, buf_hbm.at[s0_v], sem)
  c1 = pltpu.async_copy(rows_v, buf_hbm.at[s1_v], sem)
  c0.wait()
  c1.wait()


def _dispatch(ids_flat, w_flat, xpk):
  mesh = plsc.VectorSubcoreMesh(core_axis_name="c", subcore_axis_name="s")
  return pl.kernel(
      _dispatch_body,
      out_type=[jax.ShapeDtypeStruct((NBUF, DH), jnp.int32),
                jax.ShapeDtypeStruct((T,), jnp.int32),
                jax.ShapeDtypeStruct((T,), jnp.int32),
                jax.ShapeDtypeStruct((T,), jnp.float32),
                jax.ShapeDtypeStruct((T,), jnp.float32),
                jax.ShapeDtypeStruct((E,), jnp.int32),
                jax.ShapeDtypeStruct((E * NB,), jnp.int32)],
      mesh=mesh,
      compiler_params=pltpu.CompilerParams(needs_layout_passes=False),
      scratch_types=[pltpu.VMEM((TK,), jnp.int32),
                     pltpu.VMEM((E,), jnp.int32),
                     pltpu.VMEM((E * NB,), jnp.int32),
                     pltpu.VMEM((CHUNK,), jnp.int32),
                     pltpu.VMEM((CHUNK,), jnp.float32),
                     pltpu.VMEM((CHUNK,), jnp.int32),
                     pltpu.VMEM((CHUNK,), jnp.int32),
                     pltpu.VMEM((TPW,), jnp.int32),
                     pltpu.VMEM((TPW,), jnp.int32),
                     pltpu.VMEM((TPW,), jnp.float32),
                     pltpu.VMEM((TPW,), jnp.float32),
                     pltpu.VMEM((TPW,), jnp.int32),
                     pltpu.VMEM((TPW,), jnp.int32),
                     pltpu.VMEM((TPW, DH), jnp.int32),
                     pltpu.VMEM((E,), jnp.int32),
                     pltpu.SemaphoreType.DMA],
  )(ids_flat, w_flat, xpk)


def _mlp_body(pf_ref, counts_ref, buf_ref, wgu_ref, wdn_ref, out_ref):
  del pf_ref
  e = pl.program_id(0)
  b = pl.program_id(1)
  cnt = counts_ref[e]

  @pl.when(cnt > b * BM)
  def _():
    xlo, xhi = _unpack_halves(buf_ref[...])
    wgu = wgu_ref[0]
    gu = (jnp.dot(xlo, wgu[:DH].astype(jnp.bfloat16),
                  preferred_element_type=jnp.float32)
          + jnp.dot(xhi, wgu[DH:].astype(jnp.bfloat16),
                    preferred_element_type=jnp.float32))
    g = gu[:, :F]
    u = gu[:, F:]
    h = (g * jax.nn.sigmoid(g) * u).astype(jnp.bfloat16)
    wd = wdn_ref[0].astype(jnp.bfloat16)
    out = jnp.dot(h, wd, preferred_element_type=jnp.float32)
    out_ref[...] = _pack_halves(out[:, :DH], out[:, DH:])


def _expert_mlp(pf, counts, buf, w_gate_up, w_down):
  grid_spec = pltpu.PrefetchScalarGridSpec(
      num_scalar_prefetch=2,
      grid=(E, NB),
      in_specs=[
          pl.BlockSpec((BM, DH), lambda e, b, pf, cnt: (pf[e * NB + b], 0)),
          pl.BlockSpec((1, D, 2 * F), lambda e, b, pf, cnt: (e, 0, 0)),
          pl.BlockSpec((1, F, D), lambda e, b, pf, cnt: (e, 0, 0)),
      ],
      out_specs=pl.BlockSpec((BM, DH),
                             lambda e, b, pf, cnt: (pf[e * NB + b], 0)),
  )
  return pl.pallas_call(
      _mlp_body,
      grid_spec=grid_spec,
      out_shape=jax.ShapeDtypeStruct((TBMAX * BM, DH), jnp.int32),
      compiler_params=pltpu.CompilerParams(
          dimension_semantics=("arbitrary", "arbitrary")),
  )(pf, counts, buf, w_gate_up, w_down)


def _gather_body(oute_hbm, d0_hbm, d1_hbm, g0_hbm, g1_hbm,
                 idx0_v, idx1_v, rows0_v, rows1_v, sem0, sem1):
  wid = lax.axis_index("s") * 2 + lax.axis_index("c")
  tbase = wid * TPW
  pltpu.sync_copy(d0_hbm.at[pl.ds(tbase, TPW)], idx0_v)
  pltpu.sync_copy(d1_hbm.at[pl.ds(tbase, TPW)], idx1_v)
  c0 = pltpu.async_copy(oute_hbm.at[idx0_v], rows0_v, sem0)
  c1 = pltpu.async_copy(oute_hbm.at[idx1_v], rows1_v, sem1)
  c0.wait()
  w0 = pltpu.async_copy(rows0_v, g0_hbm.at[pl.ds(tbase, TPW)], sem0)
  c1.wait()
  w1 = pltpu.async_copy(rows1_v, g1_hbm.at[pl.ds(tbase, TPW)], sem1)
  w0.wait()
  w1.wait()


def _combine_gather(out_e, d0, d1):
  mesh = plsc.VectorSubcoreMesh(core_axis_name="c", subcore_axis_name="s")
  return pl.kernel(
      _gather_body,
      out_type=[jax.ShapeDtypeStruct((T, DH), jnp.int32),
                jax.ShapeDtypeStruct((T, DH), jnp.int32)],
      mesh=mesh,
      compiler_params=pltpu.CompilerParams(needs_layout_passes=False),
      scratch_types=[pltpu.VMEM((TPW,), jnp.int32),
                     pltpu.VMEM((TPW,), jnp.int32),
                     pltpu.VMEM((TPW, DH), jnp.int32),
                     pltpu.VMEM((TPW, DH), jnp.int32),
                     pltpu.SemaphoreType.DMA,
                     pltpu.SemaphoreType.DMA],
  )(out_e, d0, d1)


def _shared_body(x_ref, wsgu_ref, wsdn_ref, g0_ref, g1_ref,
                 wf0_ref, wf1_ref, out_ref):
  xlo, xhi = _unpack_halves(x_ref[...])
  wsgu = wsgu_ref[...]
  gu = (jnp.dot(xlo, wsgu[:DH].astype(jnp.bfloat16),
                preferred_element_type=jnp.float32)
        + jnp.dot(xhi, wsgu[DH:].astype(jnp.bfloat16),
                  preferred_element_type=jnp.float32))
  g = gu[:, :FS]
  u = gu[:, FS:]
  h = (g * jax.nn.sigmoid(g) * u).astype(jnp.bfloat16)
  wsdn = wsdn_ref[...].astype(jnp.bfloat16)
  shared = jnp.dot(h, wsdn, preferred_element_type=jnp.float32)
  g0lo, g0hi = _unpack_halves(g0_ref[...])
  g1lo, g1hi = _unpack_halves(g1_ref[...])
  wf0 = wf0_ref[...]
  wf1 = wf1_ref[...]
  olo = (shared[:, :DH] + wf0 * g0lo.astype(jnp.float32)
         + wf1 * g1lo.astype(jnp.float32))
  ohi = (shared[:, DH:] + wf0 * g0hi.astype(jnp.float32)
         + wf1 * g1hi.astype(jnp.float32))
  out_ref[...] = jnp.concatenate([olo, ohi], axis=1)


def _shared_combine(xpk, ws_gate_up, ws_down, g0, g1, wf0, wf1):
  nb = T // BMS
  return pl.pallas_call(
      _shared_body,
      grid=(nb,),
      in_specs=[
          pl.BlockSpec((BMS, DH), lambda i: (i, 0)),
          pl.BlockSpec((D, 2 * FS), lambda i: (0, 0)),
          pl.BlockSpec((FS, D), lambda i: (0, 0)),
          pl.BlockSpec((BMS, DH), lambda i: (i, 0)),
          pl.BlockSpec((BMS, DH), lambda i: (i, 0)),
          pl.BlockSpec((BMS, 1), lambda i: (i, 0)),
          pl.BlockSpec((BMS, 1), lambda i: (i, 0)),
      ],
      out_specs=pl.BlockSpec((BMS, D), lambda i: (i, 0)),
      out_shape=jax.ShapeDtypeStruct((T, D), jnp.float32),
      compiler_params=pltpu.CompilerParams(
          dimension_semantics=("arbitrary",)),
  )(xpk, ws_gate_up, ws_down, g0, g1, wf0, wf1)


@jax.jit
def kernel(hidden_states, gate_w, w_gate_up, w_down, ws_gate_up, ws_down):
  topk_w, topk_idx, xpk = _router(hidden_states, gate_w)
  ids_flat = topk_idx.reshape(TK)
  w_flat = topk_w.reshape(TK)
  buf, d0, d1, wf0, wf1, counts, pf = _dispatch(ids_flat, w_flat, xpk)
  out_e = _expert_mlp(pf, counts, buf, w_gate_up, w_down)
  g0, g1 = _combine_gather(out_e, d0, d1)
  return _shared_combine(xpk, ws_gate_up, ws_down, g0, g1,
                         wf0.reshape(T, 1), wf1.reshape(T, 1))


# speedup vs baseline: 1.0129x; 1.0129x over previous
import jax
import jax.numpy as jnp
from jax import lax
from jax.experimental import pallas as pl
from jax.experimental.pallas import tpu as pltpu
from jax.experimental.pallas import tpu_sc as plsc

T = 2048
D = 1024
DH = D // 2
E = 64
KTOP = 2
F = 512
FS = 1024
CAP = 256
TK = T * KTOP

NW = 32
CHUNK = TK // NW
TPW = T // NW
BM = 128
BM_LOG = 7
NB = CAP // BM
TBMAX = E + TK // BM
TRASH = TBMAX * BM
NBUF = (TBMAX + 1) * BM
BMS = 256

_HIMASK = -65536


def _pack_halves(a, b):
  lo = pltpu.bitcast(a.astype(jnp.bfloat16).astype(jnp.float32), jnp.int32)
  hi = pltpu.bitcast(b.astype(jnp.bfloat16).astype(jnp.float32), jnp.int32)
  return (hi & _HIMASK) | lax.shift_right_logical(lo, 16)


def _unpack_halves(w):
  lo = pltpu.bitcast(lax.shift_left(w, 16), jnp.float32)
  hi = pltpu.bitcast(w & _HIMASK, jnp.float32)
  return lo.astype(jnp.bfloat16), hi.astype(jnp.bfloat16)


def _router_body(x_ref, gw_ref, w_ref, i_ref, xp_ref):
  x = x_ref[...]
  gw = gw_ref[...]
  xp_ref[...] = _pack_halves(x[:, :DH], x[:, DH:])
  logits = lax.dot_general(x, gw, (((1,), (1,)), ((), ())),
                           preferred_element_type=jnp.float32)
  s = jax.nn.sigmoid(logits)
  col = lax.broadcasted_iota(jnp.int32, s.shape, 1)
  m1 = jnp.max(s, axis=1, keepdims=True)
  i1 = jnp.min(jnp.where(s == m1, col, E), axis=1, keepdims=True)
  s2 = jnp.where(col == i1, -1e30, s)
  m2 = jnp.max(s2, axis=1, keepdims=True)
  i2 = jnp.min(jnp.where(s2 == m2, col, E), axis=1, keepdims=True)
  tot = m1 + m2
  w_ref[...] = jnp.concatenate([m1 / tot, m2 / tot], axis=1)
  i_ref[...] = jnp.concatenate([i1, i2], axis=1)


def _router(x, gate_w):
  nb = 4
  br = T // nb
  return pl.pallas_call(
      _router_body,
      grid=(nb,),
      in_specs=[pl.BlockSpec((br, D), lambda i: (i, 0)),
                pl.BlockSpec((E, D), lambda i: (0, 0))],
      out_specs=[pl.BlockSpec((br, KTOP), lambda i: (i, 0)),
                 pl.BlockSpec((br, KTOP), lambda i: (i, 0)),
                 pl.BlockSpec((br, DH), lambda i: (i, 0))],
      out_shape=[jax.ShapeDtypeStruct((T, KTOP), jnp.float32),
                 jax.ShapeDtypeStruct((T, KTOP), jnp.int32),
                 jax.ShapeDtypeStruct((T, DH), jnp.int32)],
      compiler_params=pltpu.CompilerParams(
          dimension_semantics=("arbitrary",)),
  )(x, gate_w)


def _scan_base():
  c0, _ = plsc.scan_count(jnp.zeros((16,), jnp.int32))
  return jnp.min(c0)


def _rank_body(ids_hbm, rank_hbm, hists_hbm, ids_v, rank_v, counts_v):
  wid = lax.axis_index("s") * 2 + lax.axis_index("c")
  base = wid * CHUNK
  pltpu.sync_copy(ids_hbm.at[pl.ds(base, CHUNK)], ids_v)
  for t in range(E // 16):
    counts_v[pl.ds(t * 16, 16)] = jnp.zeros((16,), jnp.int32)
  cbase = _scan_base()
  for g in range(CHUNK // 16):
    ids = ids_v[pl.ds(g * 16, 16)]
    c, last = plsc.scan_count(ids)
    prior = c - cbase
    cnt = plsc.load_gather(counts_v, [ids])
    rank = cnt + prior
    rank_v[pl.ds(g * 16, 16)] = rank
    plsc.store_scatter(counts_v, [ids], rank + 1, mask=last)
  pltpu.sync_copy(rank_v, rank_hbm.at[pl.ds(base, CHUNK)])
  pltpu.sync_copy(counts_v, hists_hbm.at[wid])


def _rank_kernel(ids_flat):
  mesh = plsc.VectorSubcoreMesh(core_axis_name="c", subcore_axis_name="s")
  return pl.kernel(
      _rank_body,
      out_type=[jax.ShapeDtypeStruct((TK,), jnp.int32),
                jax.ShapeDtypeStruct((NW, E), jnp.int32)],
      mesh=mesh,
      compiler_params=pltpu.CompilerParams(needs_layout_passes=False),
      scratch_types=[pltpu.VMEM((CHUNK,), jnp.int32),
                     pltpu.VMEM((CHUNK,), jnp.int32),
                     pltpu.VMEM((E,), jnp.int32)],
  )(ids_flat)


def _dispatch_body(ids_hbm, w_hbm, rank_hbm, hists_hbm, x_hbm,
                   buf_hbm, d0_hbm, d1_hbm, wf0_hbm, wf1_hbm, counts_hbm,
                   pf_hbm,
                   hists_v, prefix_v, blkb_v, pf_v, ids_v, rank_v, wv_v,
                   dstg_v, dsts_v, d0_v, d1_v, wf0_v, wf1_v,
                   s0_v, s1_v, rows_v, tot_v, sem):
  wid = lax.axis_index("s") * 2 + lax.axis_index("c")
  base = wid * CHUNK
  tbase = wid * TPW
  pltpu.sync_copy(hists_hbm, hists_v)
  pltpu.sync_copy(ids_hbm.at[pl.ds(base, CHUNK)], ids_v)
  pltpu.sync_copy(rank_hbm.at[pl.ds(base, CHUNK)], rank_v)
  pltpu.sync_copy(w_hbm.at[pl.ds(base, CHUNK)], wv_v)

  carry = jnp.int32(0)
  for eg in range(E // 16):
    pre = jnp.zeros((16,), jnp.int32)
    tot = jnp.zeros((16,), jnp.int32)
    for w2 in range(NW):
      row = hists_v[w2, pl.ds(eg * 16, 16)]
      pre = pre + jnp.where(w2 < wid, row, 0)
      tot = tot + row
    prefix_v[pl.ds(eg * 16, 16)] = pre
    tot_v[pl.ds(eg * 16, 16)] = tot
    act = jnp.maximum(
        1, lax.shift_right_logical(jnp.minimum(tot, CAP) + (BM - 1), BM_LOG))
    inc = plsc.cumsum(act)
    excl = inc - act + carry
    blkb_v[pl.ds(eg * 16, 16)] = excl
    carry = carry + jnp.max(inc)
    epos = (lax.iota(jnp.int32, 16) + eg * 16) * NB
    plsc.store_scatter(pf_v, [epos], excl)
    plsc.store_scatter(pf_v, [epos + 1], excl + jnp.minimum(act - 1, 1))

  @pl.when(wid == NW - 1)
  def _():
    pltpu.sync_copy(tot_v, counts_hbm)
    pltpu.sync_copy(pf_v, pf_hbm)

  for g in range(CHUNK // 16):
    ids = ids_v[pl.ds(g * 16, 16)]
    rank = rank_v[pl.ds(g * 16, 16)] + plsc.load_gather(prefix_v, [ids])
    valid = rank < CAP
    rbase = plsc.load_gather(blkb_v, [ids]) * BM
    dstg_v[pl.ds(g * 16, 16)] = rbase + jnp.where(valid, rank, 0)
    dsts_v[pl.ds(g * 16, 16)] = jnp.where(valid, rbase + rank, TRASH)
    wf = jnp.where(valid, wv_v[pl.ds(g * 16, 16)], 0.0)
    wv_v[pl.ds(g * 16, 16)] = wf

  for g in range(TPW // 16):
    pidx = (lax.iota(jnp.int32, 16) + g * 16) * 2
    d0_v[pl.ds(g * 16, 16)] = plsc.load_gather(dstg_v, [pidx])
    d1_v[pl.ds(g * 16, 16)] = plsc.load_gather(dstg_v, [pidx + 1])
    wf0_v[pl.ds(g * 16, 16)] = plsc.load_gather(wv_v, [pidx])
    wf1_v[pl.ds(g * 16, 16)] = plsc.load_gather(wv_v, [pidx + 1])
    s0_v[pl.ds(g * 16, 16)] = plsc.load_gather(dsts_v, [pidx])
    s1_v[pl.ds(g * 16, 16)] = plsc.load_gather(dsts_v, [pidx + 1])
  pltpu.sync_copy(d0_v, d0_hbm.at[pl.ds(tbase, TPW)])
  pltpu.sync_copy(d1_v, d1_hbm.at[pl.ds(tbase, TPW)])
  pltpu.sync_copy(wf0_v, wf0_hbm.at[pl.ds(tbase, TPW)])
  pltpu.sync_copy(wf1_v, wf1_hbm.at[pl.ds(tbase, TPW)])

  pltpu.sync_copy(x_hbm.at[pl.ds(tbase, TPW)], rows_v)
  c0 = pltpu.async_copy(rows_v, buf_hbm.at[s0_v], sem)
  c1 = pltpu.async_copy(rows_v, buf_hbm.at[s1_v], sem)
  c0.wait()
  c1.wait()


def _dispatch(ids_flat, w_flat, local_rank, hists, xpk):
  mesh = plsc.VectorSubcoreMesh(core_axis_name="c", subcore_axis_name="s")
  return pl.kernel(
      _dispatch_body,
      out_type=[jax.ShapeDtypeStruct((NBUF, DH), jnp.int32),
                jax.ShapeDtypeStruct((T,), jnp.int32),
                jax.ShapeDtypeStruct((T,), jnp.int32),
                jax.ShapeDtypeStruct((T,), jnp.float32),
                jax.ShapeDtypeStruct((T,), jnp.float32),
                jax.ShapeDtypeStruct((E,), jnp.int32),
                jax.ShapeDtypeStruct((E * NB,), jnp.int32)],
      mesh=mesh,
      compiler_params=pltpu.CompilerParams(needs_layout_passes=False),
      scratch_types=[pltpu.VMEM((NW, E), jnp.int32),
                     pltpu.VMEM((E,), jnp.int32),
                     pltpu.VMEM((E,), jnp.int32),
                     pltpu.VMEM((E * NB,), jnp.int32),
                     pltpu.VMEM((CHUNK,), jnp.int32),
                     pltpu.VMEM((CHUNK,), jnp.int32),
                     pltpu.VMEM((CHUNK,), jnp.float32),
                     pltpu.VMEM((CHUNK,), jnp.int32),
                     pltpu.VMEM((CHUNK,), jnp.int32),
                     pltpu.VMEM((TPW,), jnp.int32),
                     pltpu.VMEM((TPW,), jnp.int32),
                     pltpu.VMEM((TPW,), jnp.float32),
                     pltpu.VMEM((TPW,), jnp.float32),
                     pltpu.VMEM((TPW,), jnp.int32),
                     pltpu.VMEM((TPW,), jnp.int32),
                     pltpu.VMEM((TPW, DH), jnp.int32),
                     pltpu.VMEM((E,), jnp.int32),
                     pltpu.SemaphoreType.DMA],
  )(ids_flat, w_flat, local_rank, hists, xpk)


def _mlp_body(pf_ref, counts_ref, buf_ref, wgu_ref, wdn_ref, out_ref):
  del pf_ref
  e = pl.program_id(0)
  b = pl.program_id(1)
  cnt = counts_ref[e]

  @pl.when(cnt > b * BM)
  def _():
    xlo, xhi = _unpack_halves(buf_ref[...])
    wgu = wgu_ref[0]
    gu = (jnp.dot(xlo, wgu[:DH].astype(jnp.bfloat16),
                  preferred_element_type=jnp.float32)
          + jnp.dot(xhi, wgu[DH:].astype(jnp.bfloat16),
                    preferred_element_type=jnp.float32))
    g = gu[:, :F]
    u = gu[:, F:]
    h = (g * jax.nn.sigmoid(g) * u).astype(jnp.bfloat16)
    wd = wdn_ref[0].astype(jnp.bfloat16)
    out = jnp.dot(h, wd, preferred_element_type=jnp.float32)
    out_ref[...] = _pack_halves(out[:, :DH], out[:, DH:])


def _expert_mlp(pf, counts, buf, w_gate_up, w_down):
  grid_spec = pltpu.PrefetchScalarGridSpec(
      num_scalar_prefetch=2,
      grid=(E, NB),
      in_specs=[
          pl.BlockSpec((BM, DH), lambda e, b, pf, cnt: (pf[e * NB + b], 0)),
          pl.BlockSpec((1, D, 2 * F), lambda e, b, pf, cnt: (e, 0, 0)),
          pl.BlockSpec((1, F, D), lambda e, b, pf, cnt: (e, 0, 0)),
      ],
      out_specs=pl.BlockSpec((BM, DH),
                             lambda e, b, pf, cnt: (pf[e * NB + b], 0)),
  )
  return pl.pallas_call(
      _mlp_body,
      grid_spec=grid_spec,
      out_shape=jax.ShapeDtypeStruct((TBMAX * BM, DH), jnp.int32),
      compiler_params=pltpu.CompilerParams(
          dimension_semantics=("arbitrary", "arbitrary")),
  )(pf, counts, buf, w_gate_up, w_down)


def _gather_body(oute_hbm, d0_hbm, d1_hbm, g0_hbm, g1_hbm,
                 idx0_v, idx1_v, rows0_v, rows1_v, sem0, sem1):
  wid = lax.axis_index("s") * 2 + lax.axis_index("c")
  tbase = wid * TPW
  pltpu.sync_copy(d0_hbm.at[pl.ds(tbase, TPW)], idx0_v)
  pltpu.sync_copy(d1_hbm.at[pl.ds(tbase, TPW)], idx1_v)
  c0 = pltpu.async_copy(oute_hbm.at[idx0_v], rows0_v, sem0)
  c1 = pltpu.async_copy(oute_hbm.at[idx1_v], rows1_v, sem1)
  c0.wait()
  w0 = pltpu.async_copy(rows0_v, g0_hbm.at[pl.ds(tbase, TPW)], sem0)
  c1.wait()
  w1 = pltpu.async_copy(rows1_v, g1_hbm.at[pl.ds(tbase, TPW)], sem1)
  w0.wait()
  w1.wait()


def _combine_gather(out_e, d0, d1):
  mesh = plsc.VectorSubcoreMesh(core_axis_name="c", subcore_axis_name="s")
  return pl.kernel(
      _gather_body,
      out_type=[jax.ShapeDtypeStruct((T, DH), jnp.int32),
                jax.ShapeDtypeStruct((T, DH), jnp.int32)],
      mesh=mesh,
      compiler_params=pltpu.CompilerParams(needs_layout_passes=False),
      scratch_types=[pltpu.VMEM((TPW,), jnp.int32),
                     pltpu.VMEM((TPW,), jnp.int32),
                     pltpu.VMEM((TPW, DH), jnp.int32),
                     pltpu.VMEM((TPW, DH), jnp.int32),
                     pltpu.SemaphoreType.DMA,
                     pltpu.SemaphoreType.DMA],
  )(out_e, d0, d1)


def _shared_body(x_ref, wsgu_ref, wsdn_ref, g0_ref, g1_ref,
                 wf0_ref, wf1_ref, out_ref):
  xlo, xhi = _unpack_halves(x_ref[...])
  wsgu = wsgu_ref[...]
  gu = (jnp.dot(xlo, wsgu[:DH].astype(jnp.bfloat16),
                preferred_element_type=jnp.float32)
        + jnp.dot(xhi, wsgu[DH:].astype(jnp.bfloat16),
                  preferred_element_type=jnp.float32))
  g = gu[:, :FS]
  u = gu[:, FS:]
  h = (g * jax.nn.sigmoid(g) * u).astype(jnp.bfloat16)
  wsdn = wsdn_ref[...].astype(jnp.bfloat16)
  shared = jnp.dot(h, wsdn, preferred_element_type=jnp.float32)
  g0lo, g0hi = _unpack_halves(g0_ref[...])
  g1lo, g1hi = _unpack_halves(g1_ref[...])
  wf0 = wf0_ref[...]
  wf1 = wf1_ref[...]
  olo = (shared[:, :DH] + wf0 * g0lo.astype(jnp.float32)
         + wf1 * g1lo.astype(jnp.float32))
  ohi = (shared[:, DH:] + wf0 * g0hi.astype(jnp.float32)
         + wf1 * g1hi.astype(jnp.float32))
  out_ref[...] = jnp.concatenate([olo, ohi], axis=1)


def _shared_combine(xpk, ws_gate_up, ws_down, g0, g1, wf0, wf1):
  nb = T // BMS
  return pl.pallas_call(
      _shared_body,
      grid=(nb,),
      in_specs=[
          pl.BlockSpec((BMS, DH), lambda i: (i, 0)),
          pl.BlockSpec((D, 2 * FS), lambda i: (0, 0)),
          pl.BlockSpec((FS, D), lambda i: (0, 0)),
          pl.BlockSpec((BMS, DH), lambda i: (i, 0)),
          pl.BlockSpec((BMS, DH), lambda i: (i, 0)),
          pl.BlockSpec((BMS, 1), lambda i: (i, 0)),
          pl.BlockSpec((BMS, 1), lambda i: (i, 0)),
      ],
      out_specs=pl.BlockSpec((BMS, D), lambda i: (i, 0)),
      out_shape=jax.ShapeDtypeStruct((T, D), jnp.float32),
      compiler_params=pltpu.CompilerParams(
          dimension_semantics=("arbitrary",)),
  )(xpk, ws_gate_up, ws_down, g0, g1, wf0, wf1)


@jax.jit
def kernel(hidden_states, gate_w, w_gate_up, w_down, ws_gate_up, ws_down):
  topk_w, topk_idx, xpk = _router(hidden_states, gate_w)
  ids_flat = topk_idx.reshape(TK)
  w_flat = topk_w.reshape(TK)
  local_rank, hists = _rank_kernel(ids_flat)
  buf, d0, d1, wf0, wf1, counts, pf = _dispatch(
      ids_flat, w_flat, local_rank, hists, xpk)
  out_e = _expert_mlp(pf, counts, buf, w_gate_up, w_down)
  g0, g1 = _combine_gather(out_e, d0, d1)
  return _shared_combine(xpk, ws_gate_up, ws_down, g0, g1,
                         wf0.reshape(T, 1), wf1.reshape(T, 1))
